# Initial kernel scaffold; baseline (speedup 1.0000x reference)
import jax
import jax.numpy as jnp
from jax.experimental import pallas as pl


def kernel(inputs, edge_index, edge_attr, W1, a_s1, a_d1, b1, W2, a_s2, a_d2, b2, W3, a_s3, a_d3, b3, Wm, bm, Wv, bv, Wd1, bd1, Wd2, bd2):
    raise NotImplementedError("write your pallas kernel here")


# speedup vs baseline: 19.6212x; 19.6212x over previous
import functools

import jax
import jax.numpy as jnp
from jax import lax
from jax.experimental import pallas as pl
from jax.experimental.pallas import tpu as pltpu
from jax.experimental.pallas import tpu_sc as plsc

N = 10000
NP = 10240
E = 160000

NCORES = 2
NSUB = 16
NW = NCORES * NSUB
EPW = E // NW
CH = 128
NFULL = EPW // CH
TAIL = EPW - NFULL * CH
RPT = NP // NSUB
ZR = 128

_f32 = jnp.float32


def _sc_edge_weights(asr_p, adt_p, src, dst):
    mesh = plsc.VectorSubcoreMesh(core_axis_name="c", subcore_axis_name="s")

    @functools.partial(
        pl.kernel, mesh=mesh,
        compiler_params=pltpu.CompilerParams(use_tc_tiling_on_sc=False),
        out_type=[jax.ShapeDtypeStruct((E, 16), _f32),
                  jax.ShapeDtypeStruct((NCORES * NP, 16), _f32)],
        scratch_types=[
            pltpu.VMEM((CH,), jnp.int32), pltpu.VMEM((CH,), jnp.int32),
            pltpu.VMEM((CH, 16), _f32), pltpu.VMEM((CH, 16), _f32),
            pltpu.VMEM((CH, 16), _f32),
            pltpu.VMEM((TAIL,), jnp.int32), pltpu.VMEM((TAIL,), jnp.int32),
            pltpu.VMEM((TAIL, 16), _f32), pltpu.VMEM((TAIL, 16), _f32),
            pltpu.VMEM((TAIL, 16), _f32),
            pltpu.VMEM((ZR, 16), _f32),
            pltpu.VMEM_SHARED((NP, 16), _f32),
            pltpu.SemaphoreType.DMA, pltpu.SemaphoreType.DMA,
        ],
    )
    def k(asr_h, adt_h, src_h, dst_h, w_h, den_h,
          src_v, dst_v, a_v, b_v, w_v,
          src_t, dst_t, a_t, b_t, w_t,
          z_v, den_sh, sem1, sem2):
        c = lax.axis_index("c")
        s = lax.axis_index("s")
        wid = s * NCORES + c

        def zrow(i, _):
            z_v[i] = jnp.zeros((16,), _f32)
            return 0
        lax.fori_loop(0, ZR, zrow, 0)
        for q in range(RPT // ZR):
            pltpu.sync_copy(z_v, den_sh.at[pl.ds(s * RPT + q * ZR, ZR)])
        plsc.subcore_barrier()

        def do_chunk(base, n, srcv, dstv, av, bv, wv):
            pltpu.sync_copy(src_h.at[pl.ds(base, n)], srcv)
            pltpu.sync_copy(dst_h.at[pl.ds(base, n)], dstv)
            pltpu.async_copy(asr_h.at[srcv], av, sem1).wait()
            pltpu.async_copy(adt_h.at[dstv], bv, sem2).wait()

            def row(j, _):
                e = av[j] + bv[j]
                e = jnp.where(e > 0, e, 0.2 * e)
                wv[j] = jnp.exp(e)
                return 0
            lax.fori_loop(0, n, row, 0)
            pltpu.sync_copy(wv, w_h.at[pl.ds(base, n)])
            pltpu.sync_copy(wv, den_sh.at[dstv], add=True)

        def chunk(kk, _):
            do_chunk(wid * EPW + kk * CH, CH, src_v, dst_v, a_v, b_v, w_v)
            return 0
        lax.fori_loop(0, NFULL, chunk, 0)
        do_chunk(wid * EPW + NFULL * CH, TAIL, src_t, dst_t, a_t, b_t, w_t)

        plsc.subcore_barrier()
        pltpu.sync_copy(den_sh.at[pl.ds(s * RPT, RPT)],
                        den_h.at[pl.ds(c * NP + s * RPT, RPT)])

    return k(asr_p, adt_p, src, dst)


def _sc_weighted_agg(hg, w, src, dst, lane0, hpg, cc):
    F = hpg * cc
    mesh = plsc.VectorSubcoreMesh(core_axis_name="c", subcore_axis_name="s")

    @functools.partial(
        pl.kernel, mesh=mesh,
        compiler_params=pltpu.CompilerParams(use_tc_tiling_on_sc=False),
        out_type=jax.ShapeDtypeStruct((NCORES * NP, F), _f32),
        scratch_types=[
            pltpu.VMEM((CH,), jnp.int32), pltpu.VMEM((CH,), jnp.int32),
            pltpu.VMEM((CH, 16), _f32), pltpu.VMEM((CH, F), _f32),
            pltpu.VMEM((TAIL,), jnp.int32), pltpu.VMEM((TAIL,), jnp.int32),
            pltpu.VMEM((TAIL, 16), _f32), pltpu.VMEM((TAIL, F), _f32),
            pltpu.VMEM((ZR, F), _f32),
            pltpu.VMEM_SHARED((NP, F), _f32),
            pltpu.SemaphoreType.DMA,
        ],
    )
    def k(hg_h, w_h, src_h, dst_h, out_h,
          src_v, dst_v, w_v, h_v,
          src_t, dst_t, w_t, h_t,
          z_v, acc_sh, sem1):
        c = lax.axis_index("c")
        s = lax.axis_index("s")
        wid = s * NCORES + c

        def zrow(i, _):
            for q in range(F // 16):
                z_v[i, pl.ds(q * 16, 16)] = jnp.zeros((16,), _f32)
            return 0
        lax.fori_loop(0, ZR, zrow, 0)
        for q in range(RPT // ZR):
            pltpu.sync_copy(z_v, acc_sh.at[pl.ds(s * RPT + q * ZR, ZR)])
        plsc.subcore_barrier()

        def do_chunk(base, n, srcv, dstv, wv, hv):
            pltpu.sync_copy(src_h.at[pl.ds(base, n)], srcv)
            pltpu.sync_copy(dst_h.at[pl.ds(base, n)], dstv)
            pltpu.sync_copy(w_h.at[pl.ds(base, n)], wv)
            pltpu.async_copy(hg_h.at[srcv], hv, sem1).wait()

            def row(j, _):
                wrow = wv[j]
                for hh in range(hpg):
                    wb = jnp.full((16,), wrow[lane0 + hh], _f32)
                    for q in range(cc // 16):
                        col = hh * cc + q * 16
                        hv[j, pl.ds(col, 16)] = hv[j, pl.ds(col, 16)] * wb
                return 0
            lax.fori_loop(0, n, row, 0)
            pltpu.sync_copy(hv, acc_sh.at[dstv], add=True)

        def chunk(kk, _):
            do_chunk(wid * EPW + kk * CH, CH, src_v, dst_v, w_v, h_v)
            return 0
        lax.fori_loop(0, NFULL, chunk, 0)
        do_chunk(wid * EPW + NFULL * CH, TAIL, src_t, dst_t, w_t, h_t)

        plsc.subcore_barrier()
        pltpu.sync_copy(acc_sh.at[pl.ds(s * RPT, RPT)],
                        out_h.at[pl.ds(c * NP + s * RPT, RPT)])

    return k(hg, w, src, dst)


def _sc_scatter_rows(table, src, dst):
    mesh = plsc.VectorSubcoreMesh(core_axis_name="c", subcore_axis_name="s")

    @functools.partial(
        pl.kernel, mesh=mesh,
        compiler_params=pltpu.CompilerParams(use_tc_tiling_on_sc=False),
        out_type=jax.ShapeDtypeStruct((NCORES * NP, 16), _f32),
        scratch_types=[
            pltpu.VMEM((CH,), jnp.int32), pltpu.VMEM((CH,), jnp.int32),
            pltpu.VMEM((CH, 16), _f32),
            pltpu.VMEM((TAIL,), jnp.int32), pltpu.VMEM((TAIL,), jnp.int32),
            pltpu.VMEM((TAIL, 16), _f32),
            pltpu.VMEM((ZR, 16), _f32),
            pltpu.VMEM_SHARED((NP, 16), _f32),
            pltpu.SemaphoreType.DMA,
        ],
    )
    def k(tab_h, src_h, dst_h, out_h,
          src_v, dst_v, t_v, src_t, dst_t, t_t, z_v, acc_sh, sem1):
        c = lax.axis_index("c")
        s = lax.axis_index("s")
        wid = s * NCORES + c

        def zrow(i, _):
            z_v[i] = jnp.zeros((16,), _f32)
            return 0
        lax.fori_loop(0, ZR, zrow, 0)
        for q in range(RPT // ZR):
            pltpu.sync_copy(z_v, acc_sh.at[pl.ds(s * RPT + q * ZR, ZR)])
        plsc.subcore_barrier()

        def do_chunk(base, n, srcv, dstv, tv):
            pltpu.sync_copy(src_h.at[pl.ds(base, n)], srcv)
            pltpu.sync_copy(dst_h.at[pl.ds(base, n)], dstv)
            pltpu.async_copy(tab_h.at[srcv], tv, sem1).wait()
            pltpu.sync_copy(tv, acc_sh.at[dstv], add=True)

        def chunk(kk, _):
            do_chunk(wid * EPW + kk * CH, CH, src_v, dst_v, t_v)
            return 0
        lax.fori_loop(0, NFULL, chunk, 0)
        do_chunk(wid * EPW + NFULL * CH, TAIL, src_t, dst_t, t_t)

        plsc.subcore_barrier()
        pltpu.sync_copy(acc_sh.at[pl.ds(s * RPT, RPT)],
                        out_h.at[pl.ds(c * NP + s * RPT, RPT)])

    return k(table, src, dst)


_TB = 2048


def _tc_pre(x, W, As, Ad, G, Fg):
    Din = x.shape[1]
    Dout = W.shape[1]
    grid = NP // _TB

    def body(x_ref, W_ref, As_ref, Ad_ref, *refs):
        hg_refs = refs[:G]
        asr_ref, adt_ref, ws_ref = refs[G:]
        h = jnp.dot(x_ref[...], W_ref[...], preferred_element_type=_f32)
        for g in range(G):
            hg_refs[g][...] = h[:, g * Fg:(g + 1) * Fg]
        asr = jnp.dot(h, As_ref[...], preferred_element_type=_f32)
        adt = jnp.dot(h, Ad_ref[...], preferred_element_type=_f32)
        asr_ref[...] = asr
        adt_ref[...] = adt
        e = asr + adt
        e = jnp.where(e > 0, e, 0.2 * e)
        ws_ref[...] = jnp.exp(e)

    out_shape = ([jax.ShapeDtypeStruct((NP, Fg), _f32) for _ in range(G)]
                 + [jax.ShapeDtypeStruct((NP, 16), _f32)] * 3)
    out_specs = ([pl.BlockSpec((_TB, Fg), lambda i: (i, 0)) for _ in range(G)]
                 + [pl.BlockSpec((_TB, 16), lambda i: (i, 0))] * 3)
    res = pl.pallas_call(
        body,
        grid=(grid,),
        in_specs=[pl.BlockSpec((_TB, Din), lambda i: (i, 0)),
                  pl.BlockSpec((Din, Dout), lambda i: (0, 0)),
                  pl.BlockSpec((Dout, 16), lambda i: (0, 0)),
                  pl.BlockSpec((Dout, 16), lambda i: (0, 0))],
        out_specs=out_specs,
        out_shape=out_shape,
    )(x, W, As, Ad)
    return res[:G], res[G], res[G + 1], res[G + 2]


def _tc_post(ogs, den, wself, hgs, b, H, Cc, relu):
    G = len(ogs)
    hpg = H // G
    F = hpg * Cc
    grid = NP // _TB

    def body(*refs):
        og_refs = refs[:G]
        den_ref = refs[G]
        ws_ref = refs[G + 1]
        hg_refs = refs[G + 2:2 * G + 2]
        b_ref = refs[2 * G + 2]
        o_ref = refs[2 * G + 3]
        ws = ws_ref[...]
        den_t = den_ref[0] + den_ref[1] + ws
        acc = None
        for h in range(H):
            g, hh = divmod(h, hpg)
            c0 = hh * Cc
            num = (og_refs[g][0, :, c0:c0 + Cc] + og_refs[g][1, :, c0:c0 + Cc]
                   + ws[:, h:h + 1] * hg_refs[g][:, c0:c0 + Cc])
            t = num / (den_t[:, h:h + 1] + 1e-16)
            acc = t if acc is None else acc + t
        out = acc / H + b_ref[...]
        if relu:
            out = jnp.maximum(out, 0.0)
        o_ref[...] = out

    in_specs = ([pl.BlockSpec((2, _TB, F), lambda i: (0, i, 0)) for _ in range(G)]
                + [pl.BlockSpec((2, _TB, 16), lambda i: (0, i, 0)),
                   pl.BlockSpec((_TB, 16), lambda i: (i, 0))]
                + [pl.BlockSpec((_TB, F), lambda i: (i, 0)) for _ in range(G)]
                + [pl.BlockSpec((1, Cc), lambda i: (0, 0))])
    return pl.pallas_call(
        body,
        grid=(grid,),
        in_specs=in_specs,
        out_specs=pl.BlockSpec((_TB, Cc), lambda i: (i, 0)),
        out_shape=jax.ShapeDtypeStruct((NP, Cc), _f32),
    )(*ogs, den, wself, *hgs, b)


def _tc_heads(z, Wm, bm, Wv, bv, Wd1, bd1, Wd2, bd2):
    grid = NP // _TB

    def body(z_ref, Wm_ref, bm_ref, Wv_ref, bv_ref, Wd1_ref, bd1_ref,
             Wd2_ref, bd2_ref, rf_ref, kl_ref):
        z = z_ref[...]
        zm = jnp.dot(z, Wm_ref[...], preferred_element_type=_f32) + bm_ref[...]
        zv = jnp.clip(jnp.exp(jnp.dot(z, Wv_ref[...],
                                      preferred_element_type=_f32) + bv_ref[...]),
                      1e-08, 100.0)
        t = jnp.maximum(jnp.dot(z, Wd1_ref[...],
                                preferred_element_type=_f32) + bd1_ref[...], 0.0)
        rf = jnp.dot(t, Wd2_ref[...], preferred_element_type=_f32) + bd2_ref[...]
        rf_ref[...] = rf
        sigma = jnp.sqrt(zv)
        kl = jnp.sum(-jnp.log(sigma) + 0.5 * (sigma ** 2 + zm ** 2 - 1.0),
                     axis=1, keepdims=True) * 0.5
        kl_ref[...] = kl

    def vspec(r, c):
        return pl.BlockSpec((r, c), lambda i: (0, 0))

    return pl.pallas_call(
        body,
        grid=(grid,),
        in_specs=[pl.BlockSpec((_TB, 32), lambda i: (i, 0)),
                  vspec(32, 32), vspec(1, 32), vspec(32, 32), vspec(1, 32),
                  vspec(32, 64), vspec(1, 64), vspec(64, 128), vspec(1, 128)],
        out_specs=[pl.BlockSpec((_TB, 128), lambda i: (i, 0)),
                   pl.BlockSpec((_TB, 1), lambda i: (i, 0))],
        out_shape=[jax.ShapeDtypeStruct((NP, 128), _f32),
                   jax.ShapeDtypeStruct((NP, 1), _f32)],
    )(z, Wm, bm, Wv, bv, Wd1, bd1, Wd2, bd2)


_NPAD = 10240


def _tc_mcmc(rf, st0, cands, us):
    def body(rf_ref, st0_ref, cand_ref, u_ref, p_ref):
        rf = rf_ref[...]
        sm = jnp.mean(rf, axis=1, keepdims=True)
        smm = jnp.mean(sm)
        sstd = jnp.sqrt(jnp.mean((sm - smm) ** 2))
        smn = (sm - smm) / (sstd + 1e-08)
        Ls = []
        for si in range(3):
            mv = jnp.mean(rf[si, :])
            Ls.append(jnp.sum(jnp.exp(-0.5 * (smn - mv) ** 2)))
        pr = [jnp.float32(0.1), jnp.float32(0.8), jnp.float32(0.1)]
        T = [[jnp.float32(0.98), jnp.float32(0.01), jnp.float32(0.01)],
             [jnp.float32(0.01), jnp.float32(0.98), jnp.float32(0.01)],
             [jnp.float32(0.01), jnp.float32(0.01), jnp.float32(0.98)]]
        R = [[(pr[j] / pr[i]) * (Ls[j] / Ls[i]) * (T[i][j] / T[j][i])
              for j in range(3)] for i in range(3)]

        states = st0_ref[...]
        p0 = jnp.zeros(states.shape, _f32)
        p1 = jnp.zeros(states.shape, _f32)
        p2 = jnp.zeros(states.shape, _f32)
        for it in range(20):
            cand = cand_ref[it]
            uu = u_ref[it]
            ratio = jnp.zeros(states.shape, _f32)
            for i in range(3):
                for j in range(3):
                    m = (states == i) & (cand == j)
                    ratio = jnp.where(m, R[i][j], ratio)
            states = jnp.where(uu < ratio, cand, states)
            p0 = p0 + (states == 0).astype(_f32)
            p1 = p1 + (states == 1).astype(_f32)
            p2 = p2 + (states == 2).astype(_f32)
        p_ref[0] = p0
        p_ref[1] = p1
        p_ref[2] = p2

    return pl.pallas_call(
        body,
        in_specs=[pl.BlockSpec((N, 128), lambda: (0, 0)),
                  pl.BlockSpec((_NPAD // 128, 128), lambda: (0, 0)),
                  pl.BlockSpec((20, _NPAD // 128, 128), lambda: (0, 0, 0)),
                  pl.BlockSpec((20, _NPAD // 128, 128), lambda: (0, 0, 0))],
        out_specs=pl.BlockSpec((3, _NPAD // 128, 128), lambda: (0, 0, 0)),
        out_shape=jax.ShapeDtypeStruct((3, _NPAD // 128, 128), _f32),
    )(rf, st0, cands, us)


def _tc_final(rf, x0, spp):
    def body(rf_ref, x0_ref, spp_ref, nc_ref, reg_ref):
        rf = rf_ref[...]
        x0 = x0_ref[...]
        sp = spp_ref[0] + spp_ref[1]
        cnt = jnp.maximum(sp[:, 3:4], 1.0)
        a0 = sp[:, 0:1] / cnt
        a1 = sp[:, 1:2] / cnt
        a2 = sp[:, 2:3] / cnt
        st = jnp.where((a0 >= a1) & (a0 >= a2), 1.0,
                       jnp.where(a1 >= a2, 2.0, 3.0))
        sac = rf * st
        nc = sac / (jnp.sum(sac, axis=1, keepdims=True) + 1e-08) \
            * jnp.sum(x0, axis=1, keepdims=True)
        mn = jnp.min(nc)
        mx = jnp.max(nc)
        rmin = mn * 0.8
        rmax = mx * 1.2
        nc = (nc - mn) / (mx - mn + 1e-08)
        nc = nc * (rmax - rmin) + rmin
        nc = nc / jnp.mean(nc)
        nc_ref[...] = nc
        reg_ref[...] = jnp.reshape(jnp.sum(rf ** 2) * jnp.float32(0.0001), (1, 1))

    return pl.pallas_call(
        body,
        in_specs=[pl.BlockSpec((N, 128), lambda: (0, 0)),
                  pl.BlockSpec((N, 128), lambda: (0, 0)),
                  pl.BlockSpec((2, N, 16), lambda: (0, 0, 0))],
        out_specs=[pl.BlockSpec((N, 128), lambda: (0, 0)),
                   pl.BlockSpec((1, 1), lambda: (0, 0))],
        out_shape=[jax.ShapeDtypeStruct((N, 128), _f32),
                   jax.ShapeDtypeStruct((1, 1), _f32)],
    )(rf, x0, spp)


def _pack_attn(a):
    H, C = a.shape
    M = jnp.zeros((H * C, 16), _f32)
    rows = jnp.arange(H * C)
    cols = jnp.repeat(jnp.arange(H), C)
    return M.at[rows, cols].set(a.reshape(-1))


def _gat_layer(x, W, a_s, a_d, b, src, dst, relu):
    H, Cc = a_s.shape
    G = max(H // 2, 1)
    hpg = H // G
    Fg = hpg * Cc
    hgs, asr_p, adt_p, wself = _tc_pre(x, W, _pack_attn(a_s), _pack_attn(a_d),
                                       G, Fg)
    w, den = _sc_edge_weights(asr_p, adt_p, src, dst)
    den = den.reshape(2, NP, 16)
    ogs = [
        _sc_weighted_agg(hgs[g], w, src, dst, g * hpg, hpg, Cc).reshape(2, NP, Fg)
        for g in range(G)
    ]
    return _tc_post(ogs, den, wself, hgs, b.reshape(1, Cc), H, Cc, relu)


def kernel(inputs, edge_index, edge_attr, W1, a_s1, a_d1, b1, W2, a_s2, a_d2,
           b2, W3, a_s3, a_d3, b3, Wm, bm, Wv, bv, Wd1, bd1, Wd2, bd2):
    src = edge_index[0].astype(jnp.int32)
    dst = edge_index[1].astype(jnp.int32)

    x0 = jnp.pad(inputs, ((0, NP - N), (0, 0)))
    x1 = _gat_layer(x0, W1, a_s1, a_d1, b1, src, dst, True)
    x2 = _gat_layer(x1, W2, a_s2, a_d2, b2, src, dst, True)
    z = _gat_layer(x2, W3, a_s3, a_d3, b3, src, dst, False)

    rf_p, kl_p = _tc_heads(z, Wm, bm.reshape(1, 32), Wv, bv.reshape(1, 32),
                           Wd1, bd1.reshape(1, 64), Wd2, bd2.reshape(1, 128))
    rf = rf_p[:N]
    kl = kl_p[:N]

    key = jax.random.key(42)
    st0 = jax.random.randint(jax.random.fold_in(key, 999), (N,), 0, 3)
    cands = jnp.stack([jax.random.randint(jax.random.fold_in(key, 2 * it),
                                          (N,), 0, 3) for it in range(20)])
    us = jnp.stack([jax.random.uniform(jax.random.fold_in(key, 2 * it + 1),
                                       (N,)) for it in range(20)])
    pad = _NPAD - N
    st0 = jnp.pad(st0, (0, pad)).reshape(_NPAD // 128, 128)
    cands = jnp.pad(cands, ((0, 0), (0, pad))).reshape(20, _NPAD // 128, 128)
    us = jnp.pad(us, ((0, 0), (0, pad))).reshape(20, _NPAD // 128, 128)

    probs3 = _tc_mcmc(rf, st0, cands, us)
    probs = probs3.reshape(3, _NPAD)[:, :N].T
    probs_p = jnp.concatenate(
        [probs, jnp.ones((N, 1), _f32), jnp.zeros((N, 12), _f32)], axis=1)
    probs_p = jnp.pad(probs_p, ((0, NP - N), (0, 0)))

    spp = _sc_scatter_rows(probs_p, src, dst).reshape(2, NP, 16)[:, :N]
    nc, reg = _tc_final(rf, inputs, spp)

    return nc, rf, kl.reshape(N), reg.reshape(())
